# baseline (speedup 1.0000x reference)
import functools

import jax
import jax.numpy as jnp
from jax import lax
from jax.experimental import pallas as pl
from jax.experimental.pallas import tpu as pltpu
from jax.experimental.pallas import tpu_sc as plsc

N = 10000
E = 320000
F_IN = 128
H = 64
B = 64

EP = E + N
NC = 2
NS = 16
NW = NC * NS
CHUNK = 128
NCHUNK = 2 * (-(-EP // (NW * CHUNK * 2)))
EPT = NCHUNK * CHUNK
EPAD = EPT * NW
WROW = 80
NR = 624
NRLAST = N - 15 * NR
RZ = 208

_f32 = jnp.float32
_i32 = jnp.int32


def _tc_lin_body(x_ref, wl_ref, wr_ref, xl_ref, xr_ref):
    xb = x_ref[...].astype(jnp.bfloat16)
    xl_ref[...] = jnp.dot(xb, wl_ref[...].astype(jnp.bfloat16),
                          preferred_element_type=_f32)
    xr_ref[...] = jnp.dot(xb, wr_ref[...].astype(jnp.bfloat16),
                          preferred_element_type=_f32)


def _tc_lin(x, wl, wr):
    n, f = x.shape
    blk = 1000
    return pl.pallas_call(
        _tc_lin_body,
        grid=(n // blk,),
        in_specs=[
            pl.BlockSpec((blk, f), lambda i: (i, 0)),
            pl.BlockSpec((f, H), lambda i: (0, 0)),
            pl.BlockSpec((f, H), lambda i: (0, 0)),
        ],
        out_specs=[
            pl.BlockSpec((blk, H), lambda i: (i, 0)),
            pl.BlockSpec((blk, H), lambda i: (i, 0)),
        ],
        out_shape=[
            jax.ShapeDtypeStruct((n, H), _f32),
            jax.ShapeDtypeStruct((n, H), _f32),
        ],
    )(x, wl, wr)


def _sc_edge_body(xl_hbm, xr_hbm, src_hbm, dst_hbm, att_hbm, out_hbm,
                  att_v, sidxa, didxa, sidxb, didxb, ua, va, ub, vb, wa, wb,
                  zbuf, acc_sp, ga1, ga2, gb1, gb2, sca, scb):
    c = lax.axis_index("c")
    sid = lax.axis_index("s")
    wid = sid * NC + c

    iota = lax.iota(_i32, 16)
    cols = [iota + 16 * k for k in range(5)]

    def zrow(j, carry):
        rowj = jnp.full((16,), j, _i32)
        zero = jnp.zeros((16,), _f32)
        for k in range(5):
            plsc.store_scatter(zbuf, [rowj, cols[k]], zero)
        return carry

    lax.fori_loop(0, RZ, zrow, 0)
    r0 = sid * NR
    for q in range(NR // RZ):
        pltpu.sync_copy(zbuf, acc_sp.at[pl.ds(r0 + q * RZ, RZ)])

    @pl.when(sid == NS - 1)
    def _zero_tail():
        for q in range(-(-(NRLAST - NR) // RZ)):
            nrows = min(RZ, NRLAST - NR - q * RZ)
            pltpu.sync_copy(zbuf.at[pl.ds(0, nrows)],
                            acc_sp.at[pl.ds(15 * NR + NR + q * RZ, nrows)])

    plsc.subcore_barrier()

    pltpu.sync_copy(att_hbm, att_v)
    att_regs = [att_v[pl.ds(16 * k, 16)] for k in range(4)]

    def prefetch(t, sidx, didx, u, v, g1, g2):
        base = wid * EPT + t * CHUNK
        pltpu.sync_copy(src_hbm.at[pl.ds(base, CHUNK)], sidx)
        pltpu.sync_copy(dst_hbm.at[pl.ds(base, CHUNK)], didx)
        pltpu.async_copy(xl_hbm.at[sidx], u, g1)
        pltpu.async_copy(xr_hbm.at[didx], v, g2)

    def compute(t, u, v, w):
        base = wid * EPT + t * CHUNK

        @plsc.parallel_loop(0, CHUNK, unroll=6)
        def edge_body(i):
            rowi = jnp.full((16,), i, _i32)
            us = [plsc.load_gather(u, [rowi, cols[k]]) for k in range(4)]
            acc = jnp.zeros((16,), _f32)
            for k in range(4):
                vk = plsc.load_gather(v, [rowi, cols[k]])
                h = us[k] + vk
                lr = jnp.maximum(h, 0.0) + 0.2 * jnp.minimum(h, 0.0)
                lb = lax.bitcast_convert_type(lr, _i32)
                lb = lb + 32767 + ((lb >> 16) & 1)
                lr = lax.bitcast_convert_type(lb & _i32(-65536), _f32)
                acc = acc + att_regs[k] * lr
            tot = jnp.sum(acc)
            ok = (base + i) < EP
            scale = jnp.where(ok, 1.0, 0.0).astype(_f32)
            sb = jnp.exp(jnp.broadcast_to(tot, (16,))) * scale
            for k in range(4):
                plsc.store_scatter(w, [rowi, cols[k]], us[k] * sb)
            plsc.store_scatter(w, [rowi, cols[4]], sb)

    JMAX = NCHUNK // 2
    prefetch(0, sidxa, didxa, ua, va, ga1, ga2)
    prefetch(1, sidxb, didxb, ub, vb, gb1, gb2)

    def pair_body(j, carry):
        for (t0, sidx, didx, u, v, w, g1, g2, sc) in (
            (2 * j, sidxa, didxa, ua, va, wa, ga1, ga2, sca),
            (2 * j + 1, sidxb, didxb, ub, vb, wb, gb1, gb2, scb),
        ):
            pltpu.make_async_copy(xl_hbm.at[sidx], u, g1).wait()
            pltpu.make_async_copy(xr_hbm.at[didx], v, g2).wait()

            @pl.when(j > 0)
            def _wait_prev_scatter():
                pltpu.make_async_copy(w, acc_sp.at[didx], sc).wait()

            compute(t0, u, v, w)
            pltpu.async_copy(w, acc_sp.at[didx], sc, add=True)

            @pl.when(j < JMAX - 1)
            def _next():
                prefetch(t0 + 2, sidx, didx, u, v, g1, g2)

        return carry

    lax.fori_loop(0, JMAX, pair_body, 0)
    pltpu.make_async_copy(wa, acc_sp.at[didxa], sca).wait()
    pltpu.make_async_copy(wb, acc_sp.at[didxb], scb).wait()

    plsc.subcore_barrier()
    pltpu.sync_copy(acc_sp.at[pl.ds(r0, NR)], out_hbm.at[c, pl.ds(r0, NR)])

    @pl.when(sid == NS - 1)
    def _dump_tail():
        t0 = 16 * NR
        pltpu.sync_copy(acc_sp.at[pl.ds(t0, NRLAST - NR)],
                        out_hbm.at[c, pl.ds(t0, NRLAST - NR)])


def _sc_edge(xl, xr, src, dst, att):
    mesh = plsc.VectorSubcoreMesh(core_axis_name="c", subcore_axis_name="s")
    fn = pl.kernel(
        _sc_edge_body,
        out_type=jax.ShapeDtypeStruct((NC, N, WROW), _f32),
        mesh=mesh,
        compiler_params=pltpu.CompilerParams(
            needs_layout_passes=False, use_tc_tiling_on_sc=False),
        scratch_types=[
            pltpu.VMEM((H,), _f32),
            pltpu.VMEM((CHUNK,), _i32),
            pltpu.VMEM((CHUNK,), _i32),
            pltpu.VMEM((CHUNK,), _i32),
            pltpu.VMEM((CHUNK,), _i32),
            pltpu.VMEM((CHUNK, H), _f32),
            pltpu.VMEM((CHUNK, H), _f32),
            pltpu.VMEM((CHUNK, H), _f32),
            pltpu.VMEM((CHUNK, H), _f32),
            pltpu.VMEM((CHUNK, WROW), _f32),
            pltpu.VMEM((CHUNK, WROW), _f32),
            pltpu.VMEM((RZ, WROW), _f32),
            pltpu.VMEM_SHARED((N, WROW), _f32),
            pltpu.SemaphoreType.DMA,
            pltpu.SemaphoreType.DMA,
            pltpu.SemaphoreType.DMA,
            pltpu.SemaphoreType.DMA,
            pltpu.SemaphoreType.DMA,
            pltpu.SemaphoreType.DMA,
        ],
    )
    return fn(xl, xr, src, dst, att)


def _tc_mid_body(acc_ref, b_ref, wl_ref, wr_ref, xl_ref, xr_ref):
    acc = acc_ref[...]
    msg = acc[0, :, :H] + acc[1, :, :H]
    den = acc[0, :, H] + acc[1, :, H]
    hid = jnp.maximum(msg / den[:, None] + b_ref[...], 0.0)
    hidb = hid.astype(jnp.bfloat16)
    xl_ref[...] = jnp.dot(hidb, wl_ref[...].astype(jnp.bfloat16),
                          preferred_element_type=_f32)
    xr_ref[...] = jnp.dot(hidb, wr_ref[...].astype(jnp.bfloat16),
                          preferred_element_type=_f32)


def _tc_mid(acc, bvec, wl, wr):
    blk = 1000
    return pl.pallas_call(
        _tc_mid_body,
        grid=(N // blk,),
        in_specs=[
            pl.BlockSpec((NC, blk, WROW), lambda i: (0, i, 0)),
            pl.BlockSpec((1, H), lambda i: (0, 0)),
            pl.BlockSpec((H, H), lambda i: (0, 0)),
            pl.BlockSpec((H, H), lambda i: (0, 0)),
        ],
        out_specs=[
            pl.BlockSpec((blk, H), lambda i: (i, 0)),
            pl.BlockSpec((blk, H), lambda i: (i, 0)),
        ],
        out_shape=[
            jax.ShapeDtypeStruct((N, H), _f32),
            jax.ShapeDtypeStruct((N, H), _f32),
        ],
    )(acc, bvec, wl, wr)


def _tc_final_body(acc_ref, b_ref, batch_ref, wout_ref, bout_ref, out_ref):
    acc = acc_ref[...]
    msg = acc[0, :, :H] + acc[1, :, :H]
    den = acc[0, :, H] + acc[1, :, H]
    hid = jnp.maximum(msg / den[:, None] + b_ref[...], 0.0)
    bi = batch_ref[...]

    def body(bb, pooled):
        mask = bi == bb
        row = jnp.max(jnp.where(mask, hid, -jnp.inf), axis=0)
        sel = lax.broadcasted_iota(_i32, (B, H), 0) == bb
        return jnp.where(sel, row[None, :], pooled)

    pooled = lax.fori_loop(0, B, body, jnp.full((B, H), -jnp.inf, _f32))
    pooled = jnp.where(jnp.isfinite(pooled), pooled, 0.0)
    out_ref[...] = (
        jnp.dot(pooled.astype(jnp.bfloat16),
                wout_ref[...].astype(jnp.bfloat16),
                preferred_element_type=_f32)
        + bout_ref[...]
    )


def _tc_final(acc, bvec, batch2d, wout, bout):
    return pl.pallas_call(
        _tc_final_body,
        grid=(1,),
        in_specs=[
            pl.BlockSpec((NC, N, WROW), lambda i: (0, 0, 0)),
            pl.BlockSpec((1, H), lambda i: (0, 0)),
            pl.BlockSpec((N, 1), lambda i: (0, 0)),
            pl.BlockSpec((H, 1), lambda i: (0, 0)),
            pl.BlockSpec((1, 1), lambda i: (0, 0)),
        ],
        out_specs=pl.BlockSpec((B, 1), lambda i: (0, 0)),
        out_shape=jax.ShapeDtypeStruct((B, 1), _f32),
    )(acc, bvec, batch2d, wout, bout)


def kernel(x, edge_index, batch_index, W_l1, W_r1, att1, b1,
           W_l2, W_r2, att2, b2, W_out, b_out):
    loop = jnp.arange(N, dtype=_i32)
    padlen = EPAD - EP
    zpad = jnp.zeros((padlen,), _i32)
    src = jnp.concatenate([edge_index[0], loop, zpad])
    dst = jnp.concatenate([edge_index[1], loop, zpad])

    att1r = att1.astype(jnp.bfloat16).astype(_f32)
    att2r = att2.astype(jnp.bfloat16).astype(_f32)
    xl1, xr1 = _tc_lin(x, W_l1, W_r1)
    acc1 = _sc_edge(xl1, xr1, src, dst, att1r)
    xl2, xr2 = _tc_mid(acc1, b1.reshape(1, H), W_l2, W_r2)
    acc2 = _sc_edge(xl2, xr2, src, dst, att2r)
    out = _tc_final(acc2, b2.reshape(1, H), batch_index.reshape(N, 1),
                    W_out, b_out.reshape(1, 1))
    return out


# speedup vs baseline: 1.0079x; 1.0079x over previous
import functools

import jax
import jax.numpy as jnp
from jax import lax
from jax.experimental import pallas as pl
from jax.experimental.pallas import tpu as pltpu
from jax.experimental.pallas import tpu_sc as plsc

N = 10000
E = 320000
F_IN = 128
H = 64
B = 64

EP = E + N
NC = 2
NS = 16
NW = NC * NS
CHUNK = 128
NCHUNK = 2 * (-(-EP // (NW * CHUNK * 2)))
EPT = NCHUNK * CHUNK
EPAD = EPT * NW
WROW = 80
NR = 624
NRLAST = N - 15 * NR
RZ = 208

_f32 = jnp.float32
_i32 = jnp.int32


def _tc_lin_body(x_ref, wl_ref, wr_ref, xl_ref, xr_ref):
    xb = x_ref[...].astype(jnp.bfloat16)
    xl_ref[...] = jnp.dot(xb, wl_ref[...].astype(jnp.bfloat16),
                          preferred_element_type=_f32)
    xr_ref[...] = jnp.dot(xb, wr_ref[...].astype(jnp.bfloat16),
                          preferred_element_type=_f32)


def _tc_lin(x, wl, wr):
    n, f = x.shape
    blk = 1000
    return pl.pallas_call(
        _tc_lin_body,
        grid=(n // blk,),
        in_specs=[
            pl.BlockSpec((blk, f), lambda i: (i, 0)),
            pl.BlockSpec((f, H), lambda i: (0, 0)),
            pl.BlockSpec((f, H), lambda i: (0, 0)),
        ],
        out_specs=[
            pl.BlockSpec((blk, H), lambda i: (i, 0)),
            pl.BlockSpec((blk, H), lambda i: (i, 0)),
        ],
        out_shape=[
            jax.ShapeDtypeStruct((n, H), _f32),
            jax.ShapeDtypeStruct((n, H), _f32),
        ],
    )(x, wl, wr)


def _sc_edge_body(xl_hbm, xr_hbm, src_hbm, dst_hbm, att_hbm, out_hbm,
                  att_v, sidxa, didxa, sidxb, didxb, ua, va, ub, vb, wa, wb,
                  zbuf, acc_sp, ga1, ga2, gb1, gb2, sca, scb):
    c = lax.axis_index("c")
    sid = lax.axis_index("s")
    wid = sid * NC + c

    iota = lax.iota(_i32, 16)
    cols = [iota + 16 * k for k in range(5)]

    def zrow(j, carry):
        rowj = jnp.full((16,), j, _i32)
        zero = jnp.zeros((16,), _f32)
        for k in range(5):
            plsc.store_scatter(zbuf, [rowj, cols[k]], zero)
        return carry

    lax.fori_loop(0, RZ, zrow, 0)
    r0 = sid * NR
    for q in range(NR // RZ):
        pltpu.sync_copy(zbuf, acc_sp.at[pl.ds(r0 + q * RZ, RZ)])

    @pl.when(sid == NS - 1)
    def _zero_tail():
        for q in range(-(-(NRLAST - NR) // RZ)):
            nrows = min(RZ, NRLAST - NR - q * RZ)
            pltpu.sync_copy(zbuf.at[pl.ds(0, nrows)],
                            acc_sp.at[pl.ds(15 * NR + NR + q * RZ, nrows)])

    plsc.subcore_barrier()

    pltpu.sync_copy(att_hbm, att_v)
    att_regs = [att_v[pl.ds(16 * k, 16)] for k in range(4)]

    def prefetch(t, sidx, didx, u, v, g1, g2):
        base = wid * EPT + t * CHUNK
        pltpu.sync_copy(src_hbm.at[pl.ds(base, CHUNK)], sidx)
        pltpu.sync_copy(dst_hbm.at[pl.ds(base, CHUNK)], didx)
        pltpu.async_copy(xl_hbm.at[sidx], u, g1)
        pltpu.async_copy(xr_hbm.at[didx], v, g2)

    def compute(t, u, v, w):
        base = wid * EPT + t * CHUNK

        @plsc.parallel_loop(0, CHUNK, unroll=8)
        def edge_body(i):
            rowi = jnp.full((16,), i, _i32)
            acc = jnp.zeros((16,), _f32)
            for k in range(4):
                uk = plsc.load_gather(u, [rowi, cols[k]])
                vk = plsc.load_gather(v, [rowi, cols[k]])
                h = uk + vk
                lr = jnp.maximum(h, 0.0) + 0.2 * jnp.minimum(h, 0.0)
                lb = lax.bitcast_convert_type(lr, _i32)
                lb = lb + 32767 + ((lb >> 16) & 1)
                lr = lax.bitcast_convert_type(lb & _i32(-65536), _f32)
                acc = acc + att_regs[k] * lr
            tot = jnp.sum(acc)
            ok = (base + i) < EP
            scale = jnp.where(ok, 1.0, 0.0).astype(_f32)
            sb = jnp.exp(jnp.broadcast_to(tot, (16,))) * scale
            for k in range(4):
                uk = plsc.load_gather(u, [rowi, cols[k]])
                plsc.store_scatter(w, [rowi, cols[k]], uk * sb)
            plsc.store_scatter(w, [rowi, cols[4]], sb)

    JMAX = NCHUNK // 2
    prefetch(0, sidxa, didxa, ua, va, ga1, ga2)
    prefetch(1, sidxb, didxb, ub, vb, gb1, gb2)

    def pair_body(j, carry):
        for (t0, sidx, didx, u, v, w, g1, g2, sc) in (
            (2 * j, sidxa, didxa, ua, va, wa, ga1, ga2, sca),
            (2 * j + 1, sidxb, didxb, ub, vb, wb, gb1, gb2, scb),
        ):
            pltpu.make_async_copy(xl_hbm.at[sidx], u, g1).wait()
            pltpu.make_async_copy(xr_hbm.at[didx], v, g2).wait()

            @pl.when(j > 0)
            def _wait_prev_scatter():
                pltpu.make_async_copy(w, acc_sp.at[didx], sc).wait()

            compute(t0, u, v, w)
            pltpu.async_copy(w, acc_sp.at[didx], sc, add=True)

            @pl.when(j < JMAX - 1)
            def _next():
                prefetch(t0 + 2, sidx, didx, u, v, g1, g2)

        return carry

    lax.fori_loop(0, JMAX, pair_body, 0)
    pltpu.make_async_copy(wa, acc_sp.at[didxa], sca).wait()
    pltpu.make_async_copy(wb, acc_sp.at[didxb], scb).wait()

    plsc.subcore_barrier()
    pltpu.sync_copy(acc_sp.at[pl.ds(r0, NR)], out_hbm.at[c, pl.ds(r0, NR)])

    @pl.when(sid == NS - 1)
    def _dump_tail():
        t0 = 16 * NR
        pltpu.sync_copy(acc_sp.at[pl.ds(t0, NRLAST - NR)],
                        out_hbm.at[c, pl.ds(t0, NRLAST - NR)])


def _sc_edge(xl, xr, src, dst, att):
    mesh = plsc.VectorSubcoreMesh(core_axis_name="c", subcore_axis_name="s")
    fn = pl.kernel(
        _sc_edge_body,
        out_type=jax.ShapeDtypeStruct((NC, N, WROW), _f32),
        mesh=mesh,
        compiler_params=pltpu.CompilerParams(
            needs_layout_passes=False, use_tc_tiling_on_sc=False),
        scratch_types=[
            pltpu.VMEM((H,), _f32),
            pltpu.VMEM((CHUNK,), _i32),
            pltpu.VMEM((CHUNK,), _i32),
            pltpu.VMEM((CHUNK,), _i32),
            pltpu.VMEM((CHUNK,), _i32),
            pltpu.VMEM((CHUNK, H), _f32),
            pltpu.VMEM((CHUNK, H), _f32),
            pltpu.VMEM((CHUNK, H), _f32),
            pltpu.VMEM((CHUNK, H), _f32),
            pltpu.VMEM((CHUNK, WROW), _f32),
            pltpu.VMEM((CHUNK, WROW), _f32),
            pltpu.VMEM((RZ, WROW), _f32),
            pltpu.VMEM_SHARED((N, WROW), _f32),
            pltpu.SemaphoreType.DMA,
            pltpu.SemaphoreType.DMA,
            pltpu.SemaphoreType.DMA,
            pltpu.SemaphoreType.DMA,
            pltpu.SemaphoreType.DMA,
            pltpu.SemaphoreType.DMA,
        ],
    )
    return fn(xl, xr, src, dst, att)


def _tc_mid_body(acc_ref, b_ref, wl_ref, wr_ref, xl_ref, xr_ref):
    acc = acc_ref[...]
    msg = acc[0, :, :H] + acc[1, :, :H]
    den = acc[0, :, H] + acc[1, :, H]
    hid = jnp.maximum(msg / den[:, None] + b_ref[...], 0.0)
    hidb = hid.astype(jnp.bfloat16)
    xl_ref[...] = jnp.dot(hidb, wl_ref[...].astype(jnp.bfloat16),
                          preferred_element_type=_f32)
    xr_ref[...] = jnp.dot(hidb, wr_ref[...].astype(jnp.bfloat16),
                          preferred_element_type=_f32)


def _tc_mid(acc, bvec, wl, wr):
    blk = 1000
    return pl.pallas_call(
        _tc_mid_body,
        grid=(N // blk,),
        in_specs=[
            pl.BlockSpec((NC, blk, WROW), lambda i: (0, i, 0)),
            pl.BlockSpec((1, H), lambda i: (0, 0)),
            pl.BlockSpec((H, H), lambda i: (0, 0)),
            pl.BlockSpec((H, H), lambda i: (0, 0)),
        ],
        out_specs=[
            pl.BlockSpec((blk, H), lambda i: (i, 0)),
            pl.BlockSpec((blk, H), lambda i: (i, 0)),
        ],
        out_shape=[
            jax.ShapeDtypeStruct((N, H), _f32),
            jax.ShapeDtypeStruct((N, H), _f32),
        ],
    )(acc, bvec, wl, wr)


def _tc_final_body(acc_ref, b_ref, batch_ref, wout_ref, bout_ref, out_ref):
    acc = acc_ref[...]
    msg = acc[0, :, :H] + acc[1, :, :H]
    den = acc[0, :, H] + acc[1, :, H]
    hid = jnp.maximum(msg / den[:, None] + b_ref[...], 0.0)
    bi = batch_ref[...]

    def body(bb, pooled):
        mask = bi == bb
        row = jnp.max(jnp.where(mask, hid, -jnp.inf), axis=0)
        sel = lax.broadcasted_iota(_i32, (B, H), 0) == bb
        return jnp.where(sel, row[None, :], pooled)

    pooled = lax.fori_loop(0, B, body, jnp.full((B, H), -jnp.inf, _f32))
    pooled = jnp.where(jnp.isfinite(pooled), pooled, 0.0)
    out_ref[...] = (
        jnp.dot(pooled.astype(jnp.bfloat16),
                wout_ref[...].astype(jnp.bfloat16),
                preferred_element_type=_f32)
        + bout_ref[...]
    )


def _tc_final(acc, bvec, batch2d, wout, bout):
    return pl.pallas_call(
        _tc_final_body,
        grid=(1,),
        in_specs=[
            pl.BlockSpec((NC, N, WROW), lambda i: (0, 0, 0)),
            pl.BlockSpec((1, H), lambda i: (0, 0)),
            pl.BlockSpec((N, 1), lambda i: (0, 0)),
            pl.BlockSpec((H, 1), lambda i: (0, 0)),
            pl.BlockSpec((1, 1), lambda i: (0, 0)),
        ],
        out_specs=pl.BlockSpec((B, 1), lambda i: (0, 0)),
        out_shape=jax.ShapeDtypeStruct((B, 1), _f32),
    )(acc, bvec, batch2d, wout, bout)


def kernel(x, edge_index, batch_index, W_l1, W_r1, att1, b1,
           W_l2, W_r2, att2, b2, W_out, b_out):
    loop = jnp.arange(N, dtype=_i32)
    padlen = EPAD - EP
    zpad = jnp.zeros((padlen,), _i32)
    src = jnp.concatenate([edge_index[0], loop, zpad])
    dst = jnp.concatenate([edge_index[1], loop, zpad])

    att1r = att1.astype(jnp.bfloat16).astype(_f32)
    att2r = att2.astype(jnp.bfloat16).astype(_f32)
    xl1, xr1 = _tc_lin(x, W_l1, W_r1)
    acc1 = _sc_edge(xl1, xr1, src, dst, att1r)
    xl2, xr2 = _tc_mid(acc1, b1.reshape(1, H), W_l2, W_r2)
    acc2 = _sc_edge(xl2, xr2, src, dst, att2r)
    out = _tc_final(acc2, b2.reshape(1, H), batch_index.reshape(N, 1),
                    W_out, b_out.reshape(1, 1))
    return out


# speedup vs baseline: 1.1654x; 1.1562x over previous
import functools

import jax
import jax.numpy as jnp
from jax import lax
from jax.experimental import pallas as pl
from jax.experimental.pallas import tpu as pltpu
from jax.experimental.pallas import tpu_sc as plsc

N = 10000
E = 320000
F_IN = 128
H = 64
B = 64

EP = E + N
NC = 2
NS = 16
NW = NC * NS
CHUNK = 128
NCHUNK = 2 * (-(-EP // (NW * CHUNK * 2)))
EPT = NCHUNK * CHUNK
EPAD = EPT * NW
WROW = 80
NR = 624
NRLAST = N - 15 * NR
RZ = 208

_f32 = jnp.float32
_i32 = jnp.int32


def _tc_lin_body(x_ref, wl_ref, wr_ref, xl_ref, xr_ref):
    xb = x_ref[...].astype(jnp.bfloat16)
    xl_ref[...] = jnp.dot(xb, wl_ref[...].astype(jnp.bfloat16),
                          preferred_element_type=_f32)
    xr_ref[...] = jnp.dot(xb, wr_ref[...].astype(jnp.bfloat16),
                          preferred_element_type=_f32)


def _tc_lin(x, wl, wr):
    n, f = x.shape
    blk = 1000
    return pl.pallas_call(
        _tc_lin_body,
        grid=(n // blk,),
        in_specs=[
            pl.BlockSpec((blk, f), lambda i: (i, 0)),
            pl.BlockSpec((f, H), lambda i: (0, 0)),
            pl.BlockSpec((f, H), lambda i: (0, 0)),
        ],
        out_specs=[
            pl.BlockSpec((blk, H), lambda i: (i, 0)),
            pl.BlockSpec((blk, H), lambda i: (i, 0)),
        ],
        out_shape=[
            jax.ShapeDtypeStruct((n, H), _f32),
            jax.ShapeDtypeStruct((n, H), _f32),
        ],
    )(x, wl, wr)


def _sc_edge_body(xl_hbm, xr_hbm, src_hbm, dst_hbm, att_hbm, out_hbm,
                  att_v, sidxa, didxa, sidxb, didxb, ua, va, ub, vb, wa, wb,
                  zbuf, acc_sp, ga1, ga2, gb1, gb2, sca, scb):
    c = lax.axis_index("c")
    sid = lax.axis_index("s")
    wid = sid * NC + c

    iota = lax.iota(_i32, 16)
    cols = [iota + 16 * k for k in range(5)]

    def zrow(j, carry):
        rowj = jnp.full((16,), j, _i32)
        zero = jnp.zeros((16,), _f32)
        for k in range(5):
            plsc.store_scatter(zbuf, [rowj, cols[k]], zero)
        return carry

    lax.fori_loop(0, RZ, zrow, 0)
    r0 = sid * NR
    for q in range(NR // RZ):
        pltpu.sync_copy(zbuf, acc_sp.at[pl.ds(r0 + q * RZ, RZ)])

    @pl.when(sid == NS - 1)
    def _zero_tail():
        for q in range(-(-(NRLAST - NR) // RZ)):
            nrows = min(RZ, NRLAST - NR - q * RZ)
            pltpu.sync_copy(zbuf.at[pl.ds(0, nrows)],
                            acc_sp.at[pl.ds(15 * NR + NR + q * RZ, nrows)])

    plsc.subcore_barrier()

    pltpu.sync_copy(att_hbm, att_v)
    att_regs = [att_v[pl.ds(16 * k, 16)] for k in range(4)]

    def prefetch(t, sidx, didx, u, v, g1, g2):
        base = wid * EPT + t * CHUNK
        pltpu.sync_copy(src_hbm.at[pl.ds(base, CHUNK)], sidx)
        pltpu.sync_copy(dst_hbm.at[pl.ds(base, CHUNK)], didx)
        pltpu.async_copy(xl_hbm.at[sidx], u, g1)
        pltpu.async_copy(xr_hbm.at[didx], v, g2)

    def compute(t, u, v, w):
        base = wid * EPT + t * CHUNK

        @plsc.parallel_loop(0, CHUNK, unroll=4)
        def edge_body(i):
            rowi = jnp.full((16,), i, _i32)
            acc = jnp.zeros((16,), _f32)
            for k in range(4):
                uk = plsc.load_gather(u, [rowi, cols[k]])
                vk = plsc.load_gather(v, [rowi, cols[k]])
                h = uk + vk
                lr = jnp.maximum(h, 0.0) + 0.2 * jnp.minimum(h, 0.0)
                lb = lax.bitcast_convert_type(lr, _i32)
                lb = lb + 32767 + ((lb >> 16) & 1)
                lr = lax.bitcast_convert_type(lb & _i32(-65536), _f32)
                acc = acc + att_regs[k] * lr
            tot = jnp.sum(acc)
            ok = (base + i) < EP
            scale = jnp.where(ok, 1.0, 0.0).astype(_f32)
            sb = jnp.exp(jnp.broadcast_to(tot, (16,))) * scale
            for k in range(4):
                uk = plsc.load_gather(u, [rowi, cols[k]])
                plsc.store_scatter(w, [rowi, cols[k]], uk * sb)
            plsc.store_scatter(w, [rowi, cols[4]], sb)

    JMAX = NCHUNK // 2
    prefetch(0, sidxa, didxa, ua, va, ga1, ga2)
    prefetch(1, sidxb, didxb, ub, vb, gb1, gb2)

    def pair_body(j, carry):
        for (t0, sidx, didx, u, v, w, g1, g2, sc) in (
            (2 * j, sidxa, didxa, ua, va, wa, ga1, ga2, sca),
            (2 * j + 1, sidxb, didxb, ub, vb, wb, gb1, gb2, scb),
        ):
            pltpu.make_async_copy(xl_hbm.at[sidx], u, g1).wait()
            pltpu.make_async_copy(xr_hbm.at[didx], v, g2).wait()

            @pl.when(j > 0)
            def _wait_prev_scatter():
                pltpu.make_async_copy(w, acc_sp.at[didx], sc).wait()

            compute(t0, u, v, w)
            pltpu.async_copy(w, acc_sp.at[didx], sc, add=True)

            @pl.when(j < JMAX - 1)
            def _next():
                prefetch(t0 + 2, sidx, didx, u, v, g1, g2)

        return carry

    lax.fori_loop(0, JMAX, pair_body, 0)
    pltpu.make_async_copy(wa, acc_sp.at[didxa], sca).wait()
    pltpu.make_async_copy(wb, acc_sp.at[didxb], scb).wait()

    plsc.subcore_barrier()
    pltpu.sync_copy(acc_sp.at[pl.ds(r0, NR)], out_hbm.at[c, pl.ds(r0, NR)])

    @pl.when(sid == NS - 1)
    def _dump_tail():
        t0 = 16 * NR
        pltpu.sync_copy(acc_sp.at[pl.ds(t0, NRLAST - NR)],
                        out_hbm.at[c, pl.ds(t0, NRLAST - NR)])


def _sc_edge(xl, xr, src, dst, att):
    mesh = plsc.VectorSubcoreMesh(core_axis_name="c", subcore_axis_name="s")
    fn = pl.kernel(
        _sc_edge_body,
        out_type=jax.ShapeDtypeStruct((NC, N, WROW), _f32),
        mesh=mesh,
        compiler_params=pltpu.CompilerParams(
            needs_layout_passes=False, use_tc_tiling_on_sc=False),
        scratch_types=[
            pltpu.VMEM((H,), _f32),
            pltpu.VMEM((CHUNK,), _i32),
            pltpu.VMEM((CHUNK,), _i32),
            pltpu.VMEM((CHUNK,), _i32),
            pltpu.VMEM((CHUNK,), _i32),
            pltpu.VMEM((CHUNK, H), _f32),
            pltpu.VMEM((CHUNK, H), _f32),
            pltpu.VMEM((CHUNK, H), _f32),
            pltpu.VMEM((CHUNK, H), _f32),
            pltpu.VMEM((CHUNK, WROW), _f32),
            pltpu.VMEM((CHUNK, WROW), _f32),
            pltpu.VMEM((RZ, WROW), _f32),
            pltpu.VMEM_SHARED((N, WROW), _f32),
            pltpu.SemaphoreType.DMA,
            pltpu.SemaphoreType.DMA,
            pltpu.SemaphoreType.DMA,
            pltpu.SemaphoreType.DMA,
            pltpu.SemaphoreType.DMA,
            pltpu.SemaphoreType.DMA,
        ],
    )
    return fn(xl, xr, src, dst, att)


def _tc_mid_body(acc_ref, b_ref, wl_ref, wr_ref, xl_ref, xr_ref):
    acc = acc_ref[...]
    msg = acc[0, :, :H] + acc[1, :, :H]
    den = acc[0, :, H] + acc[1, :, H]
    hid = jnp.maximum(msg / den[:, None] + b_ref[...], 0.0)
    hidb = hid.astype(jnp.bfloat16)
    xl_ref[...] = jnp.dot(hidb, wl_ref[...].astype(jnp.bfloat16),
                          preferred_element_type=_f32)
    xr_ref[...] = jnp.dot(hidb, wr_ref[...].astype(jnp.bfloat16),
                          preferred_element_type=_f32)


def _tc_mid(acc, bvec, wl, wr):
    blk = 1000
    return pl.pallas_call(
        _tc_mid_body,
        grid=(N // blk,),
        in_specs=[
            pl.BlockSpec((NC, blk, WROW), lambda i: (0, i, 0)),
            pl.BlockSpec((1, H), lambda i: (0, 0)),
            pl.BlockSpec((H, H), lambda i: (0, 0)),
            pl.BlockSpec((H, H), lambda i: (0, 0)),
        ],
        out_specs=[
            pl.BlockSpec((blk, H), lambda i: (i, 0)),
            pl.BlockSpec((blk, H), lambda i: (i, 0)),
        ],
        out_shape=[
            jax.ShapeDtypeStruct((N, H), _f32),
            jax.ShapeDtypeStruct((N, H), _f32),
        ],
    )(acc, bvec, wl, wr)


def _tc_final_body(acc_ref, b_ref, batch_ref, wout_ref, bout_ref, out_ref):
    acc = acc_ref[...]
    msg = acc[0, :, :H] + acc[1, :, :H]
    den = acc[0, :, H] + acc[1, :, H]
    hid = jnp.maximum(msg / den[:, None] + b_ref[...], 0.0)
    bi = batch_ref[...]

    def body(bb, pooled):
        mask = bi == bb
        row = jnp.max(jnp.where(mask, hid, -jnp.inf), axis=0)
        sel = lax.broadcasted_iota(_i32, (B, H), 0) == bb
        return jnp.where(sel, row[None, :], pooled)

    pooled = lax.fori_loop(0, B, body, jnp.full((B, H), -jnp.inf, _f32))
    pooled = jnp.where(jnp.isfinite(pooled), pooled, 0.0)
    out_ref[...] = (
        jnp.dot(pooled.astype(jnp.bfloat16),
                wout_ref[...].astype(jnp.bfloat16),
                preferred_element_type=_f32)
        + bout_ref[...]
    )


def _tc_final(acc, bvec, batch2d, wout, bout):
    return pl.pallas_call(
        _tc_final_body,
        grid=(1,),
        in_specs=[
            pl.BlockSpec((NC, N, WROW), lambda i: (0, 0, 0)),
            pl.BlockSpec((1, H), lambda i: (0, 0)),
            pl.BlockSpec((N, 1), lambda i: (0, 0)),
            pl.BlockSpec((H, 1), lambda i: (0, 0)),
            pl.BlockSpec((1, 1), lambda i: (0, 0)),
        ],
        out_specs=pl.BlockSpec((B, 1), lambda i: (0, 0)),
        out_shape=jax.ShapeDtypeStruct((B, 1), _f32),
    )(acc, bvec, batch2d, wout, bout)


def kernel(x, edge_index, batch_index, W_l1, W_r1, att1, b1,
           W_l2, W_r2, att2, b2, W_out, b_out):
    loop = jnp.arange(N, dtype=_i32)
    padlen = EPAD - EP
    zpad = jnp.zeros((padlen,), _i32)
    src = jnp.concatenate([edge_index[0], loop, zpad])
    dst = jnp.concatenate([edge_index[1], loop, zpad])

    att1r = att1.astype(jnp.bfloat16).astype(_f32)
    att2r = att2.astype(jnp.bfloat16).astype(_f32)
    xl1, xr1 = _tc_lin(x, W_l1, W_r1)
    acc1 = _sc_edge(xl1, xr1, src, dst, att1r)
    xl2, xr2 = _tc_mid(acc1, b1.reshape(1, H), W_l2, W_r2)
    acc2 = _sc_edge(xl2, xr2, src, dst, att2r)
    out = _tc_final(acc2, b2.reshape(1, H), batch_index.reshape(N, 1),
                    W_out, b_out.reshape(1, 1))
    return out


# speedup vs baseline: 1.1754x; 1.0086x over previous
import functools

import jax
import jax.numpy as jnp
from jax import lax
from jax.experimental import pallas as pl
from jax.experimental.pallas import tpu as pltpu
from jax.experimental.pallas import tpu_sc as plsc

N = 10000
E = 320000
F_IN = 128
H = 64
B = 64

EP = E + N
NC = 2
NS = 16
NW = NC * NS
CHUNK = 128
NCHUNK = 2 * (-(-EP // (NW * CHUNK * 2)))
EPT = NCHUNK * CHUNK
EPAD = EPT * NW
WROW = 80
NR = 624
NRLAST = N - 15 * NR
RZ = 208

_f32 = jnp.float32
_i32 = jnp.int32


def _tc_lin_body(x_ref, wl_ref, wr_ref, xl_ref, xr_ref):
    xb = x_ref[...].astype(jnp.bfloat16)
    xl_ref[...] = jnp.dot(xb, wl_ref[...].astype(jnp.bfloat16),
                          preferred_element_type=_f32)
    xr_ref[...] = jnp.dot(xb, wr_ref[...].astype(jnp.bfloat16),
                          preferred_element_type=_f32)


def _tc_lin(x, wl, wr):
    n, f = x.shape
    blk = 1000
    return pl.pallas_call(
        _tc_lin_body,
        grid=(n // blk,),
        in_specs=[
            pl.BlockSpec((blk, f), lambda i: (i, 0)),
            pl.BlockSpec((f, H), lambda i: (0, 0)),
            pl.BlockSpec((f, H), lambda i: (0, 0)),
        ],
        out_specs=[
            pl.BlockSpec((blk, H), lambda i: (i, 0)),
            pl.BlockSpec((blk, H), lambda i: (i, 0)),
        ],
        out_shape=[
            jax.ShapeDtypeStruct((n, H), _f32),
            jax.ShapeDtypeStruct((n, H), _f32),
        ],
    )(x, wl, wr)


def _sc_edge_body(xl_hbm, xr_hbm, src_hbm, dst_hbm, att_hbm, out_hbm,
                  att_v, sidxa, didxa, sidxb, didxb, ua, va, ub, vb, wa, wb,
                  zbuf, acc_sp, ga1, ga2, gb1, gb2, sca, scb):
    c = lax.axis_index("c")
    sid = lax.axis_index("s")
    wid = sid * NC + c

    iota = lax.iota(_i32, 16)
    cols = [iota + 16 * k for k in range(5)]

    def zrow(j, carry):
        rowj = jnp.full((16,), j, _i32)
        zero = jnp.zeros((16,), _f32)
        for k in range(5):
            plsc.store_scatter(zbuf, [rowj, cols[k]], zero)
        return carry

    lax.fori_loop(0, RZ, zrow, 0)
    r0 = sid * NR
    for q in range(NR // RZ):
        pltpu.sync_copy(zbuf, acc_sp.at[pl.ds(r0 + q * RZ, RZ)])

    @pl.when(sid == NS - 1)
    def _zero_tail():
        for q in range(-(-(NRLAST - NR) // RZ)):
            nrows = min(RZ, NRLAST - NR - q * RZ)
            pltpu.sync_copy(zbuf.at[pl.ds(0, nrows)],
                            acc_sp.at[pl.ds(15 * NR + NR + q * RZ, nrows)])

    plsc.subcore_barrier()

    pltpu.sync_copy(att_hbm, att_v)
    att_regs = [att_v[pl.ds(16 * k, 16)] for k in range(4)]

    def prefetch(t, sidx, didx, u, v, g1, g2):
        base = wid * EPT + t * CHUNK
        pltpu.sync_copy(src_hbm.at[pl.ds(base, CHUNK)], sidx)
        pltpu.sync_copy(dst_hbm.at[pl.ds(base, CHUNK)], didx)
        pltpu.async_copy(xl_hbm.at[sidx], u, g1)
        pltpu.async_copy(xr_hbm.at[didx], v, g2)

    def compute(t, u, v, w):
        base = wid * EPT + t * CHUNK

        @plsc.parallel_loop(0, CHUNK, unroll=4)
        def edge_body(i):
            rowi = jnp.full((16,), i, _i32)
            us = [plsc.load_gather(u, [rowi, cols[k]]) for k in range(4)]
            acc = jnp.zeros((16,), _f32)
            for k in range(4):
                vk = plsc.load_gather(v, [rowi, cols[k]])
                h = us[k] + vk
                lr = jnp.maximum(h, 0.0) + 0.2 * jnp.minimum(h, 0.0)
                lb = lax.bitcast_convert_type(lr, _i32)
                lb = lb + 32767 + ((lb >> 16) & 1)
                lr = lax.bitcast_convert_type(lb & _i32(-65536), _f32)
                acc = acc + att_regs[k] * lr
            tot = jnp.sum(acc)
            ok = (base + i) < EP
            scale = jnp.where(ok, 1.0, 0.0).astype(_f32)
            sb = jnp.exp(jnp.broadcast_to(tot, (16,))) * scale
            for k in range(4):
                plsc.store_scatter(w, [rowi, cols[k]], us[k] * sb)
            plsc.store_scatter(w, [rowi, cols[4]], sb)

    JMAX = NCHUNK // 2
    prefetch(0, sidxa, didxa, ua, va, ga1, ga2)
    prefetch(1, sidxb, didxb, ub, vb, gb1, gb2)

    def pair_body(j, carry):
        for (t0, sidx, didx, u, v, w, g1, g2, sc) in (
            (2 * j, sidxa, didxa, ua, va, wa, ga1, ga2, sca),
            (2 * j + 1, sidxb, didxb, ub, vb, wb, gb1, gb2, scb),
        ):
            pltpu.make_async_copy(xl_hbm.at[sidx], u, g1).wait()
            pltpu.make_async_copy(xr_hbm.at[didx], v, g2).wait()

            @pl.when(j > 0)
            def _wait_prev_scatter():
                pltpu.make_async_copy(w, acc_sp.at[didx], sc).wait()

            compute(t0, u, v, w)
            pltpu.async_copy(w, acc_sp.at[didx], sc, add=True)

            @pl.when(j < JMAX - 1)
            def _next():
                prefetch(t0 + 2, sidx, didx, u, v, g1, g2)

        return carry

    lax.fori_loop(0, JMAX, pair_body, 0)
    pltpu.make_async_copy(wa, acc_sp.at[didxa], sca).wait()
    pltpu.make_async_copy(wb, acc_sp.at[didxb], scb).wait()

    plsc.subcore_barrier()
    pltpu.sync_copy(acc_sp.at[pl.ds(r0, NR)], out_hbm.at[c, pl.ds(r0, NR)])

    @pl.when(sid == NS - 1)
    def _dump_tail():
        t0 = 16 * NR
        pltpu.sync_copy(acc_sp.at[pl.ds(t0, NRLAST - NR)],
                        out_hbm.at[c, pl.ds(t0, NRLAST - NR)])


def _sc_edge(xl, xr, src, dst, att):
    mesh = plsc.VectorSubcoreMesh(core_axis_name="c", subcore_axis_name="s")
    fn = pl.kernel(
        _sc_edge_body,
        out_type=jax.ShapeDtypeStruct((NC, N, WROW), _f32),
        mesh=mesh,
        compiler_params=pltpu.CompilerParams(
            needs_layout_passes=False, use_tc_tiling_on_sc=False),
        scratch_types=[
            pltpu.VMEM((H,), _f32),
            pltpu.VMEM((CHUNK,), _i32),
            pltpu.VMEM((CHUNK,), _i32),
            pltpu.VMEM((CHUNK,), _i32),
            pltpu.VMEM((CHUNK,), _i32),
            pltpu.VMEM((CHUNK, H), _f32),
            pltpu.VMEM((CHUNK, H), _f32),
            pltpu.VMEM((CHUNK, H), _f32),
            pltpu.VMEM((CHUNK, H), _f32),
            pltpu.VMEM((CHUNK, WROW), _f32),
            pltpu.VMEM((CHUNK, WROW), _f32),
            pltpu.VMEM((RZ, WROW), _f32),
            pltpu.VMEM_SHARED((N, WROW), _f32),
            pltpu.SemaphoreType.DMA,
            pltpu.SemaphoreType.DMA,
            pltpu.SemaphoreType.DMA,
            pltpu.SemaphoreType.DMA,
            pltpu.SemaphoreType.DMA,
            pltpu.SemaphoreType.DMA,
        ],
    )
    return fn(xl, xr, src, dst, att)


def _tc_mid_body(acc_ref, b_ref, wl_ref, wr_ref, xl_ref, xr_ref):
    acc = acc_ref[...]
    msg = acc[0, :, :H] + acc[1, :, :H]
    den = acc[0, :, H] + acc[1, :, H]
    hid = jnp.maximum(msg / den[:, None] + b_ref[...], 0.0)
    hidb = hid.astype(jnp.bfloat16)
    xl_ref[...] = jnp.dot(hidb, wl_ref[...].astype(jnp.bfloat16),
                          preferred_element_type=_f32)
    xr_ref[...] = jnp.dot(hidb, wr_ref[...].astype(jnp.bfloat16),
                          preferred_element_type=_f32)


def _tc_mid(acc, bvec, wl, wr):
    blk = 1000
    return pl.pallas_call(
        _tc_mid_body,
        grid=(N // blk,),
        in_specs=[
            pl.BlockSpec((NC, blk, WROW), lambda i: (0, i, 0)),
            pl.BlockSpec((1, H), lambda i: (0, 0)),
            pl.BlockSpec((H, H), lambda i: (0, 0)),
            pl.BlockSpec((H, H), lambda i: (0, 0)),
        ],
        out_specs=[
            pl.BlockSpec((blk, H), lambda i: (i, 0)),
            pl.BlockSpec((blk, H), lambda i: (i, 0)),
        ],
        out_shape=[
            jax.ShapeDtypeStruct((N, H), _f32),
            jax.ShapeDtypeStruct((N, H), _f32),
        ],
    )(acc, bvec, wl, wr)


def _tc_final_body(acc_ref, b_ref, batch_ref, wout_ref, bout_ref, out_ref):
    acc = acc_ref[...]
    msg = acc[0, :, :H] + acc[1, :, :H]
    den = acc[0, :, H] + acc[1, :, H]
    hid = jnp.maximum(msg / den[:, None] + b_ref[...], 0.0)
    bi = batch_ref[...]

    def body(bb, pooled):
        mask = bi == bb
        row = jnp.max(jnp.where(mask, hid, -jnp.inf), axis=0)
        sel = lax.broadcasted_iota(_i32, (B, H), 0) == bb
        return jnp.where(sel, row[None, :], pooled)

    pooled = lax.fori_loop(0, B, body, jnp.full((B, H), -jnp.inf, _f32))
    pooled = jnp.where(jnp.isfinite(pooled), pooled, 0.0)
    out_ref[...] = (
        jnp.dot(pooled.astype(jnp.bfloat16),
                wout_ref[...].astype(jnp.bfloat16),
                preferred_element_type=_f32)
        + bout_ref[...]
    )


def _tc_final(acc, bvec, batch2d, wout, bout):
    return pl.pallas_call(
        _tc_final_body,
        grid=(1,),
        in_specs=[
            pl.BlockSpec((NC, N, WROW), lambda i: (0, 0, 0)),
            pl.BlockSpec((1, H), lambda i: (0, 0)),
            pl.BlockSpec((N, 1), lambda i: (0, 0)),
            pl.BlockSpec((H, 1), lambda i: (0, 0)),
            pl.BlockSpec((1, 1), lambda i: (0, 0)),
        ],
        out_specs=pl.BlockSpec((B, 1), lambda i: (0, 0)),
        out_shape=jax.ShapeDtypeStruct((B, 1), _f32),
    )(acc, bvec, batch2d, wout, bout)


def kernel(x, edge_index, batch_index, W_l1, W_r1, att1, b1,
           W_l2, W_r2, att2, b2, W_out, b_out):
    loop = jnp.arange(N, dtype=_i32)
    padlen = EPAD - EP
    zpad = jnp.zeros((padlen,), _i32)
    src = jnp.concatenate([edge_index[0], loop, zpad])
    dst = jnp.concatenate([edge_index[1], loop, zpad])

    att1r = att1.astype(jnp.bfloat16).astype(_f32)
    att2r = att2.astype(jnp.bfloat16).astype(_f32)
    xl1, xr1 = _tc_lin(x, W_l1, W_r1)
    acc1 = _sc_edge(xl1, xr1, src, dst, att1r)
    xl2, xr2 = _tc_mid(acc1, b1.reshape(1, H), W_l2, W_r2)
    acc2 = _sc_edge(xl2, xr2, src, dst, att2r)
    out = _tc_final(acc2, b2.reshape(1, H), batch_index.reshape(N, 1),
                    W_out, b_out.reshape(1, 1))
    return out
